# baseline (speedup 1.0000x reference)
import jax
import jax.numpy as jnp
from jax import lax
from jax.experimental import pallas as pl
from jax.experimental.pallas import tpu as pltpu
from jax.experimental.pallas import tpu_sc as plsc

EMB_DIM = 1000
BATCH = 16384

NUM_CORES = 2
NUM_SUBCORES = 16
LANES = 16
NUM_WORKERS = NUM_CORES * NUM_SUBCORES

COLS_PER_TILE = BATCH // NUM_WORKERS
BLOCK_COLS = 256
NUM_BLOCKS = COLS_PER_TILE // BLOCK_COLS
BLOCK_GROUPS = BLOCK_COLS // LANES
CHUNK_C = 200
NUM_SLICES = EMB_DIM // CHUNK_C
NUM_CHUNKS = NUM_BLOCKS * NUM_SLICES
DIV200_MUL = 328
DIV200_SHIFT = 16


def _scatter_phase(buf, qv, cmv, b, h, value):
  lane_iota = lax.broadcasted_iota(jnp.int32, (LANES,), 0)
  vals = jnp.full((LANES,), value, jnp.float32)
  base = b * (BLOCK_GROUPS * LANES)
  for j in range(BLOCK_GROUPS):
    q = qv[pl.ds(base + j * LANES, LANES)]
    cm = cmv[pl.ds(base + j * LANES, LANES)]
    col_idx = j * LANES + lane_iota
    plsc.store_scatter(buf, [cm, col_idx], vals, mask=(q == h))


def _one_hot_body(labels_hbm, out_hbm, labels_v, qv, cmv, buf0, buf1,
                  sem0, sem1):
  wid = lax.axis_index("s") * NUM_CORES + lax.axis_index("c")
  col_base = wid * COLS_PER_TILE

  pltpu.sync_copy(labels_hbm.at[pl.ds(col_base, COLS_PER_TILE)], labels_v)

  for g in range(COLS_PER_TILE // LANES):
    lbl = labels_v[pl.ds(g * LANES, LANES)]
    q = jax.lax.shift_right_logical(lbl * DIV200_MUL, DIV200_SHIFT)
    qv[pl.ds(g * LANES, LANES)] = q
    cmv[pl.ds(g * LANES, LANES)] = lbl - q * CHUNK_C

  bufs = (buf0, buf1)
  sems = (sem0, sem1)
  zeros16 = jnp.zeros((LANES,), jnp.float32)

  def zero_buf(buf):
    def zrow(r, _):
      for c in range(BLOCK_COLS // LANES):
        buf[r, pl.ds(c * LANES, LANES)] = zeros16
      return 0
    lax.fori_loop(0, CHUNK_C, zrow, 0)

  def dst_slice(b, h):
    return out_hbm.at[pl.ds(h * CHUNK_C, CHUNK_C),
                      pl.ds(col_base + b * BLOCK_COLS, BLOCK_COLS)]

  def start_dma(slot, b, h):
    return pltpu.async_copy(bufs[slot], dst_slice(b, h), sems[slot])

  def wait_dma(slot, b, h):
    pltpu.make_async_copy(bufs[slot], dst_slice(b, h), sems[slot]).wait()

  for t in (0, 1):
    zero_buf(bufs[t])
    _scatter_phase(bufs[t], qv, cmv, 0, t, 1.0)
    start_dma(t, 0, t)

  def body(i, _):
    for slot in (0, 1):
      t = 2 + 2 * i + slot
      b, h = t // NUM_SLICES, t % NUM_SLICES
      pb, ph = (t - 2) // NUM_SLICES, (t - 2) % NUM_SLICES
      wait_dma(slot, pb, ph)
      _scatter_phase(bufs[slot], qv, cmv, pb, ph, 0.0)
      _scatter_phase(bufs[slot], qv, cmv, b, h, 1.0)
      start_dma(slot, b, h)
    return 0
  lax.fori_loop(0, (NUM_CHUNKS - 2) // 2, body, 0)

  last = NUM_CHUNKS - 1
  wait_dma(0, (last - 1) // NUM_SLICES, (last - 1) % NUM_SLICES)
  wait_dma(1, last // NUM_SLICES, last % NUM_SLICES)


@jax.jit
def kernel(labels):
  mesh = plsc.VectorSubcoreMesh(
      core_axis_name="c", subcore_axis_name="s",
      num_cores=NUM_CORES, num_subcores=NUM_SUBCORES)
  out_t = pl.kernel(
      _one_hot_body,
      out_type=jax.ShapeDtypeStruct((EMB_DIM, BATCH), jnp.float32),
      mesh=mesh,
      scratch_types=[
          pltpu.VMEM((COLS_PER_TILE,), jnp.int32),
          pltpu.VMEM((COLS_PER_TILE,), jnp.int32),
          pltpu.VMEM((COLS_PER_TILE,), jnp.int32),
          pltpu.VMEM((CHUNK_C, BLOCK_COLS), jnp.float32),
          pltpu.VMEM((CHUNK_C, BLOCK_COLS), jnp.float32),
          pltpu.SemaphoreType.DMA,
          pltpu.SemaphoreType.DMA,
      ],
      compiler_params=pltpu.CompilerParams(
          needs_layout_passes=False, use_tc_tiling_on_sc=True),
  )(labels.astype(jnp.int32))
  return out_t.T


# speedup vs baseline: 1.0974x; 1.0974x over previous
import jax
import jax.numpy as jnp
from jax import lax
from jax.experimental import pallas as pl
from jax.experimental.pallas import tpu as pltpu
from jax.experimental.pallas import tpu_sc as plsc

EMB_DIM = 1000
BATCH = 16384

NUM_CORES = 2
NUM_SUBCORES = 16
LANES = 16
NUM_WORKERS = NUM_CORES * NUM_SUBCORES

COLS_PER_TILE = BATCH // NUM_WORKERS
BLOCK_COLS = 128
NUM_BLOCKS = COLS_PER_TILE // BLOCK_COLS
BLOCK_GROUPS = BLOCK_COLS // LANES
CHUNK_C = 200
NUM_SLICES = EMB_DIM // CHUNK_C
NUM_CHUNKS = NUM_BLOCKS * NUM_SLICES
DIV200_MUL = 328
DIV200_SHIFT = 16


def _scatter_phase(buf, qv, cmv, b, h, value):
  lane_iota = lax.broadcasted_iota(jnp.int32, (LANES,), 0)
  vals = jnp.full((LANES,), value, jnp.float32)
  base = b * (BLOCK_GROUPS * LANES)
  for j in range(BLOCK_GROUPS):
    q = qv[pl.ds(base + j * LANES, LANES)]
    cm = cmv[pl.ds(base + j * LANES, LANES)]
    col_idx = j * LANES + lane_iota
    plsc.store_scatter(buf, [cm, col_idx], vals, mask=(q == h))


def _one_hot_body(labels_hbm, out_hbm, labels_v, qv, cmv, buf0, buf1,
                  sem0, sem1, lsem):
  wid = lax.axis_index("s") * NUM_CORES + lax.axis_index("c")
  col_base = wid * COLS_PER_TILE

  labels_copy = pltpu.async_copy(
      labels_hbm.at[pl.ds(col_base, COLS_PER_TILE)], labels_v, lsem)

  bufs = (buf0, buf1)
  sems = (sem0, sem1)
  zeros16 = jnp.zeros((LANES,), jnp.float32)

  def zero_buf(buf):
    def zrow(r, _):
      for c in range(BLOCK_COLS // LANES):
        buf[r, pl.ds(c * LANES, LANES)] = zeros16
      return 0
    lax.fori_loop(0, CHUNK_C, zrow, 0)

  zero_buf(buf0)
  labels_copy.wait()

  for g in range(COLS_PER_TILE // LANES):
    lbl = labels_v[pl.ds(g * LANES, LANES)]
    q = jax.lax.shift_right_logical(lbl * DIV200_MUL, DIV200_SHIFT)
    qv[pl.ds(g * LANES, LANES)] = q
    cmv[pl.ds(g * LANES, LANES)] = lbl - q * CHUNK_C

  def dst_slice(b, h):
    return out_hbm.at[pl.ds(h * CHUNK_C, CHUNK_C),
                      pl.ds(col_base + b * BLOCK_COLS, BLOCK_COLS)]

  def start_dma(slot, b, h):
    return pltpu.async_copy(bufs[slot], dst_slice(b, h), sems[slot])

  def wait_dma(slot, b, h):
    pltpu.make_async_copy(bufs[slot], dst_slice(b, h), sems[slot]).wait()

  _scatter_phase(buf0, qv, cmv, 0, 0, 1.0)
  start_dma(0, 0, 0)
  zero_buf(buf1)
  _scatter_phase(buf1, qv, cmv, 0, 1, 1.0)
  start_dma(1, 0, 1)

  def body(i, _):
    for slot in (0, 1):
      t = 2 + 2 * i + slot
      b, h = t // NUM_SLICES, t % NUM_SLICES
      pb, ph = (t - 2) // NUM_SLICES, (t - 2) % NUM_SLICES
      wait_dma(slot, pb, ph)
      _scatter_phase(bufs[slot], qv, cmv, pb, ph, 0.0)
      _scatter_phase(bufs[slot], qv, cmv, b, h, 1.0)
      start_dma(slot, b, h)
    return 0
  lax.fori_loop(0, (NUM_CHUNKS - 2) // 2, body, 0)

  last = NUM_CHUNKS - 1
  wait_dma(0, (last - 1) // NUM_SLICES, (last - 1) % NUM_SLICES)
  wait_dma(1, last // NUM_SLICES, last % NUM_SLICES)


@jax.jit
def kernel(labels):
  mesh = plsc.VectorSubcoreMesh(
      core_axis_name="c", subcore_axis_name="s",
      num_cores=NUM_CORES, num_subcores=NUM_SUBCORES)
  out_t = pl.kernel(
      _one_hot_body,
      out_type=jax.ShapeDtypeStruct((EMB_DIM, BATCH), jnp.float32),
      mesh=mesh,
      scratch_types=[
          pltpu.VMEM((COLS_PER_TILE,), jnp.int32),
          pltpu.VMEM((COLS_PER_TILE,), jnp.int32),
          pltpu.VMEM((COLS_PER_TILE,), jnp.int32),
          pltpu.VMEM((CHUNK_C, BLOCK_COLS), jnp.float32),
          pltpu.VMEM((CHUNK_C, BLOCK_COLS), jnp.float32),
          pltpu.SemaphoreType.DMA,
          pltpu.SemaphoreType.DMA,
          pltpu.SemaphoreType.DMA,
      ],
      compiler_params=pltpu.CompilerParams(
          needs_layout_passes=False, use_tc_tiling_on_sc=True),
  )(labels.astype(jnp.int32))
  return out_t.T
